# baseline (speedup 1.0000x reference)
import functools

import jax
import jax.numpy as jnp
from jax import lax
from jax.experimental import pallas as pl
from jax.experimental.pallas import tpu as pltpu
from jax.experimental.pallas import tpu_sc as plsc

N = 20000
BOX_D = 4
SCORE_D = 3
OUT_D = BOX_D + SCORE_D
THRESHOLD = 0.5

NUM_CORES = 2
NUM_SUBCORES = 16
LANES = 16

CHUNK = 640
TAIL = N - 31 * CHUNK
TAIL_BASE = 31 * CHUNK


def _mask_cols(bv, sv, ov, ncols):
    @plsc.parallel_loop(0, ncols // LANES, 1, unroll=4)
    def group(i):
        sl = pl.ds(i * LANES, LANES)
        keep = jnp.where(sv[0, sl] >= THRESHOLD, jnp.float32(1.0), jnp.float32(0.0))
        for c in range(BOX_D):
            ov[c, sl] = bv[c, sl] * keep
        for c in range(SCORE_D):
            ov[BOX_D + c, sl] = sv[c, sl] * keep


def _sc_body(bt_hbm, st_hbm, out_hbm, bv, sv, ov, bv_t, sv_t, ov_t):
    wid = lax.axis_index("s") * NUM_CORES + lax.axis_index("c")

    @pl.when(wid < 31)
    def _main():
        base = wid * CHUNK
        pltpu.sync_copy(bt_hbm.at[:, pl.ds(base, CHUNK)], bv)
        pltpu.sync_copy(st_hbm.at[:, pl.ds(base, CHUNK)], sv)
        _mask_cols(bv, sv, ov, CHUNK)
        pltpu.sync_copy(ov, out_hbm.at[:, pl.ds(base, CHUNK)])

    @pl.when(wid == 31)
    def _tail():
        pltpu.sync_copy(bt_hbm.at[:, pl.ds(TAIL_BASE, TAIL)], bv_t)
        pltpu.sync_copy(st_hbm.at[:, pl.ds(TAIL_BASE, TAIL)], sv_t)
        _mask_cols(bv_t, sv_t, ov_t, TAIL)
        pltpu.sync_copy(ov_t, out_hbm.at[:, pl.ds(TAIL_BASE, TAIL)])


@jax.jit
def _run(bt, st):
    mesh = plsc.VectorSubcoreMesh(core_axis_name="c", subcore_axis_name="s")
    f = functools.partial(
        pl.kernel,
        out_type=jax.ShapeDtypeStruct((OUT_D, N), jnp.float32),
        mesh=mesh,
        scratch_types=[
            pltpu.VMEM((BOX_D, CHUNK), jnp.float32),
            pltpu.VMEM((SCORE_D, CHUNK), jnp.float32),
            pltpu.VMEM((OUT_D, CHUNK), jnp.float32),
            pltpu.VMEM((BOX_D, TAIL), jnp.float32),
            pltpu.VMEM((SCORE_D, TAIL), jnp.float32),
            pltpu.VMEM((OUT_D, TAIL), jnp.float32),
        ],
        compiler_params=pltpu.CompilerParams(
            needs_layout_passes=False, use_tc_tiling_on_sc=True
        ),
    )(_sc_body)
    return f(bt, st)


def kernel(boxes, scores):
    out_t = _run(boxes.T, scores.T)
    return out_t.T


# speedup vs baseline: 1.0326x; 1.0326x over previous
import functools

import jax
import jax.numpy as jnp
from jax import lax
from jax.experimental import pallas as pl
from jax.experimental.pallas import tpu as pltpu
from jax.experimental.pallas import tpu_sc as plsc

N = 20000
BOX_D = 4
SCORE_D = 3
OUT_D = BOX_D + SCORE_D
THRESHOLD = 0.5

NUM_CORES = 2
NUM_SUBCORES = 16
LANES = 16

CHUNK = 640
TAIL = N - 31 * CHUNK
TAIL_BASE = 31 * CHUNK


def _mask_cols(bv, sv, ov, ncols):
    @plsc.parallel_loop(0, ncols // LANES, 1, unroll=4)
    def group(i):
        sl = pl.ds(i * LANES, LANES)
        keep = jnp.where(sv[0, sl] >= THRESHOLD, jnp.float32(1.0), jnp.float32(0.0))
        for c in range(BOX_D):
            ov[c, sl] = bv[c, sl] * keep
        for c in range(SCORE_D):
            ov[BOX_D + c, sl] = sv[c, sl] * keep


def _sc_body(bt_hbm, st_hbm, out_hbm, bv, sv, ov, bv_t, sv_t, ov_t, sem):
    wid = lax.axis_index("s") * NUM_CORES + lax.axis_index("c")

    @pl.when(wid < 31)
    def _main():
        base = wid * CHUNK
        b_cp = pltpu.async_copy(bt_hbm.at[:, pl.ds(base, CHUNK)], bv, sem)
        s_cp = pltpu.async_copy(st_hbm.at[:, pl.ds(base, CHUNK)], sv, sem)
        b_cp.wait()
        s_cp.wait()
        _mask_cols(bv, sv, ov, CHUNK)
        pltpu.sync_copy(ov, out_hbm.at[:, pl.ds(base, CHUNK)])

    @pl.when(wid == 31)
    def _tail():
        b_cp = pltpu.async_copy(bt_hbm.at[:, pl.ds(TAIL_BASE, TAIL)], bv_t, sem)
        s_cp = pltpu.async_copy(st_hbm.at[:, pl.ds(TAIL_BASE, TAIL)], sv_t, sem)
        b_cp.wait()
        s_cp.wait()
        _mask_cols(bv_t, sv_t, ov_t, TAIL)
        pltpu.sync_copy(ov_t, out_hbm.at[:, pl.ds(TAIL_BASE, TAIL)])


@jax.jit
def _run(bt, st):
    mesh = plsc.VectorSubcoreMesh(core_axis_name="c", subcore_axis_name="s")
    f = functools.partial(
        pl.kernel,
        out_type=jax.ShapeDtypeStruct((OUT_D, N), jnp.float32),
        mesh=mesh,
        scratch_types=[
            pltpu.VMEM((BOX_D, CHUNK), jnp.float32),
            pltpu.VMEM((SCORE_D, CHUNK), jnp.float32),
            pltpu.VMEM((OUT_D, CHUNK), jnp.float32),
            pltpu.VMEM((BOX_D, TAIL), jnp.float32),
            pltpu.VMEM((SCORE_D, TAIL), jnp.float32),
            pltpu.VMEM((OUT_D, TAIL), jnp.float32),
            pltpu.SemaphoreType.DMA,
        ],
        compiler_params=pltpu.CompilerParams(
            needs_layout_passes=False, use_tc_tiling_on_sc=True
        ),
    )(_sc_body)
    return f(bt, st)


def kernel(boxes, scores):
    out_t = _run(boxes.T, scores.T)
    return out_t.T
